# baseline (speedup 1.0000x reference)
import functools

import jax
import jax.numpy as jnp
from jax import lax
from jax.experimental import pallas as pl
from jax.experimental.pallas import tpu as pltpu
from jax.experimental.pallas import tpu_sc as plsc

N = 50000
E = 800000
F_IN = 32
H = 96
L = 6
G = 2048

N_PAD = 51200
BLK = 1024
NBLK = N_PAD // BLK
PR = BLK // 4

BLKF = 1024
NBLKF = N_PAD // BLKF
PRF = BLKF // 4

_INTERPRET = False


def _tc0_body(x_ref, w_ref, m0_ref, m1_ref, m2_ref, h0_ref):
    x = x_ref[...]
    m = jnp.dot(x, w_ref[...], preferred_element_type=jnp.float32)
    m0_ref[...] = m[:, 0:32]
    m1_ref[...] = m[:, 32:64]
    m2_ref[...] = m[:, 64:96]
    h0_ref[...] = jnp.pad(x, ((0, 0), (0, H - F_IN)))


def _tc0(x, w0_in):
    return pl.pallas_call(
        _tc0_body,
        grid=(NBLK,),
        in_specs=[
            pl.BlockSpec((BLK, F_IN), lambda j: (j, 0)),
            pl.BlockSpec((F_IN, H), lambda j: (0, 0)),
        ],
        out_specs=[
            pl.BlockSpec((BLK, 32), lambda j: (j, 0)),
            pl.BlockSpec((BLK, 32), lambda j: (j, 0)),
            pl.BlockSpec((BLK, 32), lambda j: (j, 0)),
            pl.BlockSpec((BLK, H), lambda j: (j, 0)),
        ],
        out_shape=[
            jax.ShapeDtypeStruct((N_PAD, 32), jnp.float32),
            jax.ShapeDtypeStruct((N_PAD, 32), jnp.float32),
            jax.ShapeDtypeStruct((N_PAD, 32), jnp.float32),
            jax.ShapeDtypeStruct((N_PAD, H), jnp.float32),
        ],
        interpret=_INTERPRET,
    )(x, w0_in)


def _gru_block(a0p, a1p, a2p, h, w_ihT, w_hhT, b_ih, b_hh):
    agg = jnp.concatenate(
        [a0p[0] + a0p[1], a1p[0] + a1p[1], a2p[0] + a2p[1]], axis=1)
    gi = jnp.dot(agg, w_ihT, preferred_element_type=jnp.float32) + b_ih
    gh = jnp.dot(h, w_hhT, preferred_element_type=jnp.float32) + b_hh
    r = jax.nn.sigmoid(gi[:, 0:H] + gh[:, 0:H])
    z = jax.nn.sigmoid(gi[:, H:2 * H] + gh[:, H:2 * H])
    n = jnp.tanh(gi[:, 2 * H:] + r * gh[:, 2 * H:])
    return (1.0 - z) * n + z * h


def _tcb_body(a0_ref, a1_ref, a2_ref, h_ref, wihT_ref, whhT_ref, bih_ref,
              bhh_ref, wnext_ref, h1_ref, m0_ref, m1_ref, m2_ref):
    hn = _gru_block(a0_ref[...], a1_ref[...], a2_ref[...], h_ref[...],
                    wihT_ref[...], whhT_ref[...], bih_ref[...], bhh_ref[...])
    h1_ref[...] = hn
    m = jnp.dot(hn, wnext_ref[...], preferred_element_type=jnp.float32)
    m0_ref[...] = m[:, 0:32]
    m1_ref[...] = m[:, 32:64]
    m2_ref[...] = m[:, 64:96]


def _tcb(a0, a1, a2, h, w_ihT, w_hhT, b_ih, b_hh, w_next):
    part_spec = pl.BlockSpec((2, BLK, 32), lambda j: (0, j, 0))
    return pl.pallas_call(
        _tcb_body,
        grid=(NBLK,),
        in_specs=[
            part_spec, part_spec, part_spec,
            pl.BlockSpec((BLK, H), lambda j: (j, 0)),
            pl.BlockSpec((H, 3 * H), lambda j: (0, 0)),
            pl.BlockSpec((H, 3 * H), lambda j: (0, 0)),
            pl.BlockSpec((1, 3 * H), lambda j: (0, 0)),
            pl.BlockSpec((1, 3 * H), lambda j: (0, 0)),
            pl.BlockSpec((H, H), lambda j: (0, 0)),
        ],
        out_specs=[
            pl.BlockSpec((BLK, H), lambda j: (j, 0)),
            pl.BlockSpec((BLK, 32), lambda j: (j, 0)),
            pl.BlockSpec((BLK, 32), lambda j: (j, 0)),
            pl.BlockSpec((BLK, 32), lambda j: (j, 0)),
        ],
        out_shape=[
            jax.ShapeDtypeStruct((N_PAD, H), jnp.float32),
            jax.ShapeDtypeStruct((N_PAD, 32), jnp.float32),
            jax.ShapeDtypeStruct((N_PAD, 32), jnp.float32),
            jax.ShapeDtypeStruct((N_PAD, 32), jnp.float32),
        ],
        interpret=_INTERPRET,
    )(a0, a1, a2, h, w_ihT, w_hhT, b_ih, b_hh, w_next)


def _tcf_body(a0_ref, a1_ref, a2_ref, h_ref, wihT_ref, whhT_ref, bih_ref,
              bhh_ref, batch_ref, bn_s_ref, bn_t_ref, fc1T_ref, fc1b_ref,
              fc2T_ref, fc2b_ref, fc3T_ref, fc3b_ref, y_ref, acc_ref):
    j = pl.program_id(0)
    hn = _gru_block(a0_ref[...], a1_ref[...], a2_ref[...], h_ref[...],
                    wihT_ref[...], whhT_ref[...], bih_ref[...], bhh_ref[...])
    hr = jax.nn.relu(hn)

    @pl.when(j == 0)
    def _():
        acc_ref[...] = jnp.zeros_like(acc_ref)

    b = batch_ref[0, 0, :]
    gid = lax.broadcasted_iota(jnp.int32, (BLKF, G), 1)
    onehot = (b[:, None] == gid).astype(jnp.float32)
    hcat = jnp.concatenate([hr, jnp.ones((BLKF, 1), jnp.float32)], axis=1)
    acc_ref[...] += lax.dot_general(
        onehot, hcat, (((0,), (0,)), ((), ())),
        preferred_element_type=jnp.float32)

    @pl.when(j == NBLKF - 1)
    def _():
        acc = acc_ref[...]
        cnt = jnp.maximum(acc[:, H:H + 1], 1.0)
        pooled = acc[:, 0:H] / cnt
        yb = pooled * bn_s_ref[...] + bn_t_ref[...]
        y1 = jax.nn.relu(
            jnp.dot(yb, fc1T_ref[...], preferred_element_type=jnp.float32)
            + fc1b_ref[...])
        y2 = jax.nn.relu(
            jnp.dot(y1, fc2T_ref[...], preferred_element_type=jnp.float32)
            + fc2b_ref[...])
        y_ref[...] = (jnp.dot(y2, fc3T_ref[...],
                              preferred_element_type=jnp.float32)
                      + fc3b_ref[...])


def _tcf(a0, a1, a2, h, w_ihT, w_hhT, b_ih, b_hh, batch3d, bn_s, bn_t,
         fc1T, fc1b, fc2T, fc2b, fc3T, fc3b):
    part_spec = pl.BlockSpec((2, BLKF, 32), lambda j: (0, j, 0))
    full = lambda shape: pl.BlockSpec(shape, lambda j: tuple(0 for _ in shape))
    return pl.pallas_call(
        _tcf_body,
        grid=(NBLKF,),
        in_specs=[
            part_spec, part_spec, part_spec,
            pl.BlockSpec((BLKF, H), lambda j: (j, 0)),
            full((H, 3 * H)), full((H, 3 * H)), full((1, 3 * H)),
            full((1, 3 * H)),
            pl.BlockSpec((1, 1, BLKF), lambda j: (j, 0, 0)),
            full((1, H)), full((1, H)),
            full((H, 4 * H)), full((1, 4 * H)),
            full((4 * H, H)), full((1, H)),
            full((H, 1)), full((1, 1)),
        ],
        out_specs=pl.BlockSpec((G, 1), lambda j: (0, 0)),
        out_shape=jax.ShapeDtypeStruct((G, 1), jnp.float32),
        scratch_shapes=[pltpu.VMEM((G, H + 1), jnp.float32)],
        interpret=_INTERPRET,
    )(a0, a1, a2, h, w_ihT, w_hhT, b_ih, b_hh, batch3d, bn_s, bn_t,
      fc1T, fc1b, fc2T, fc2b, fc3T, fc3b)


NW = 32
CLEN = 128
NSG = 25
CHUNKS = NSG * 8
EPW = CHUNKS * CLEN
E_PAD = NW * EPW
WB = 3200
N_OUT = 16 * WB
R_ACC = N_OUT + 1024
ZROWS = R_ACC // 16


def _sc_agg_kernel(m0_hbm, m1_hbm, m2_hbm, src_hbm, dst_hbm,
                   a0_hbm, a1_hbm, a2_hbm,
                   src_ib, dst_ib, rows, zbuf, acc, isem, gsem, ssem):
    c = lax.axis_index("c")
    s = lax.axis_index("s")
    wid = c * 16 + s

    zero16 = jnp.zeros((16,), jnp.float32)

    @pl.loop(0, 64)
    def _(r):
        zbuf[r, pl.ds(0, 16)] = zero16
        zbuf[r, pl.ds(16, 16)] = zero16

    def start_idx(sg, b):
        pltpu.async_copy(src_hbm.at[wid, pl.ds(sg * 8, 8)], src_ib.at[b],
                         isem.at[b])
        pltpu.async_copy(dst_hbm.at[wid, pl.ds(sg * 8, 8)], dst_ib.at[b],
                         isem.at[b])

    def wait_idx(b):
        pltpu.make_async_copy(src_hbm.at[0, pl.ds(0, 8)], src_ib.at[b],
                              isem.at[b]).wait()
        pltpu.make_async_copy(src_hbm.at[0, pl.ds(0, 8)], dst_ib.at[b],
                              isem.at[b]).wait()

    def drain_rows_sem(sem, b):
        pltpu.make_async_copy(m0_hbm.at[pl.ds(0, 256)], rows.at[b],
                              sem.at[b]).wait()

    for p, (mg, ag) in enumerate(((m0_hbm, a0_hbm), (m1_hbm, a1_hbm),
                                  (m2_hbm, a2_hbm))):
        @pl.loop(0, ZROWS // 64)
        def _(t):
            pltpu.sync_copy(zbuf, acc.at[pl.ds(s * ZROWS + t * 64, 64)])
        plsc.subcore_barrier()

        def process_sg(sg, bI, first):
            start_idx(sg + 1, bI ^ 1)
            wait_idx(bI)
            for q in range(4):
                rb = q % 2
                if not (first and q < 2):
                    drain_rows_sem(ssem, rb)
                pltpu.async_copy(mg.at[src_ib.at[bI, 2 * q]],
                                 rows.at[rb, pl.ds(0, CLEN)], gsem.at[rb])
                pltpu.async_copy(mg.at[src_ib.at[bI, 2 * q + 1]],
                                 rows.at[rb, pl.ds(CLEN, CLEN)], gsem.at[rb])
                drain_rows_sem(gsem, rb)
                pltpu.async_copy(rows.at[rb, pl.ds(0, CLEN)],
                                 acc.at[dst_ib.at[bI, 2 * q]],
                                 ssem.at[rb], add=True)
                pltpu.async_copy(rows.at[rb, pl.ds(CLEN, CLEN)],
                                 acc.at[dst_ib.at[bI, 2 * q + 1]],
                                 ssem.at[rb], add=True)

        start_idx(0, 0)
        process_sg(0, 0, True)

        @pl.loop(0, (NSG - 1) // 2)
        def _(g):
            process_sg(1 + 2 * g, 1, False)
            process_sg(2 + 2 * g, 0, False)

        drain_rows_sem(ssem, 0)
        drain_rows_sem(ssem, 1)
        wait_idx(1)
        plsc.subcore_barrier()

        pltpu.sync_copy(acc.at[pl.ds(s * WB, WB)],
                        ag.at[c, pl.ds(s * WB, WB)])
        plsc.subcore_barrier()


def _sc_agg(m0, m1, m2, srcp, dstp):
    mesh = plsc.VectorSubcoreMesh(core_axis_name="c", subcore_axis_name="s")
    f = pl.kernel(
        _sc_agg_kernel,
        out_type=[jax.ShapeDtypeStruct((2, N_OUT, 32), jnp.float32)] * 3,
        mesh=mesh,
        scratch_types=[
            pltpu.VMEM((2, 8, CLEN), jnp.int32),
            pltpu.VMEM((2, 8, CLEN), jnp.int32),
            pltpu.VMEM((2, 2 * CLEN, 32), jnp.float32),
            pltpu.VMEM((64, 32), jnp.float32),
            pltpu.VMEM_SHARED((R_ACC, 32), jnp.float32),
            pltpu.SemaphoreType.DMA((2,)),
            pltpu.SemaphoreType.DMA((2,)),
            pltpu.SemaphoreType.DMA((2,)),
        ],
        compiler_params=pltpu.CompilerParams(use_tc_tiling_on_sc=False),
    )
    return f(m0, m1, m2, srcp, dstp)


def _edge_planes(edge_index):
    src = edge_index[0]
    dst = edge_index[1]
    pad = E_PAD - E
    fill_src = (jnp.arange(pad, dtype=jnp.int32) * 997) % N
    fill_dst = N_OUT + (jnp.arange(pad, dtype=jnp.int32) % CLEN)
    junk = jnp.zeros((NW, 8, CLEN), jnp.int32)
    srcp = jnp.concatenate([src, fill_src]).reshape(NW, CHUNKS, CLEN)
    dstp = jnp.concatenate([dst, fill_dst]).reshape(NW, CHUNKS, CLEN)
    srcp = jnp.concatenate([srcp, junk], axis=1)
    dstp = jnp.concatenate([dstp, junk], axis=1)
    return srcp, dstp


def kernel(x, edge_index, batch, ggc_weight, w_ih, w_hh, b_ih, b_hh,
           bn_gamma, bn_beta, bn_mean, bn_var,
           fc1_w, fc1_b, fc2_w, fc2_b, fc3_w, fc3_b):
    w_ihT = w_ih.T
    w_hhT = w_hh.T
    b_ih2 = b_ih[None, :]
    b_hh2 = b_hh[None, :]
    inv = bn_gamma / jnp.sqrt(bn_var + 1e-5)
    bn_s = inv[None, :]
    bn_t = (bn_beta - bn_mean * inv)[None, :]
    fc1T = fc1_w.T
    fc2T = fc2_w.T
    fc3T = fc3_w.T
    fc1b = fc1_b[None, :]
    fc2b = fc2_b[None, :]
    fc3b = fc3_b[None, :]
    x_p = jnp.pad(x, ((0, N_PAD - N), (0, 0)))
    batch_p = jnp.pad(batch, (0, N_PAD - N), constant_values=G)
    batch3d = batch_p.reshape(NBLKF, 1, BLKF)

    srcp, dstp = _edge_planes(edge_index)

    m0, m1, m2, h = _tc0(x_p, ggc_weight[0][:F_IN, :])
    for i in range(L):
        a0, a1, a2 = _sc_agg(m0, m1, m2, srcp, dstp)
        if i < L - 1:
            h, m0, m1, m2 = _tcb(a0, a1, a2, h, w_ihT, w_hhT, b_ih2, b_hh2,
                                 ggc_weight[i + 1])
        else:
            y = _tcf(a0, a1, a2, h, w_ihT, w_hhT, b_ih2, b_hh2, batch3d,
                     bn_s, bn_t, fc1T, fc1b, fc2T, fc2b, fc3T, fc3b)
    return y


# speedup vs baseline: 1.1583x; 1.1583x over previous
import functools

import jax
import jax.numpy as jnp
from jax import lax
from jax.experimental import pallas as pl
from jax.experimental.pallas import tpu as pltpu
from jax.experimental.pallas import tpu_sc as plsc

N = 50000
E = 800000
F_IN = 32
H = 96
L = 6
G = 2048

N_PAD = 51200
BLK = 1024
NBLK = N_PAD // BLK
PR = BLK // 4

BLKF = 1024
NBLKF = N_PAD // BLKF
PRF = BLKF // 4

_INTERPRET = False


def _tc0_body(x_ref, w_ref, m0_ref, m1_ref, m2_ref, h0_ref):
    x = x_ref[...]
    m = jnp.dot(x, w_ref[...], preferred_element_type=jnp.float32)
    m0_ref[...] = m[:, 0:32]
    m1_ref[...] = m[:, 32:64]
    m2_ref[...] = m[:, 64:96]
    h0_ref[...] = jnp.pad(x, ((0, 0), (0, H - F_IN)))


def _tc0(x, w0_in):
    return pl.pallas_call(
        _tc0_body,
        grid=(NBLK,),
        in_specs=[
            pl.BlockSpec((BLK, F_IN), lambda j: (j, 0)),
            pl.BlockSpec((F_IN, H), lambda j: (0, 0)),
        ],
        out_specs=[
            pl.BlockSpec((BLK, 32), lambda j: (j, 0)),
            pl.BlockSpec((BLK, 32), lambda j: (j, 0)),
            pl.BlockSpec((BLK, 32), lambda j: (j, 0)),
            pl.BlockSpec((BLK, H), lambda j: (j, 0)),
        ],
        out_shape=[
            jax.ShapeDtypeStruct((N_PAD, 32), jnp.float32),
            jax.ShapeDtypeStruct((N_PAD, 32), jnp.float32),
            jax.ShapeDtypeStruct((N_PAD, 32), jnp.float32),
            jax.ShapeDtypeStruct((N_PAD, H), jnp.float32),
        ],
        interpret=_INTERPRET,
    )(x, w0_in)


def _gru_block(a0p, a1p, a2p, h, w_ihT, w_hhT, b_ih, b_hh):
    agg = jnp.concatenate(
        [a0p[0] + a0p[1], a1p[0] + a1p[1], a2p[0] + a2p[1]], axis=1)
    gi = jnp.dot(agg, w_ihT, preferred_element_type=jnp.float32) + b_ih
    gh = jnp.dot(h, w_hhT, preferred_element_type=jnp.float32) + b_hh
    r = jax.nn.sigmoid(gi[:, 0:H] + gh[:, 0:H])
    z = jax.nn.sigmoid(gi[:, H:2 * H] + gh[:, H:2 * H])
    n = jnp.tanh(gi[:, 2 * H:] + r * gh[:, 2 * H:])
    return (1.0 - z) * n + z * h


def _tcb_body(a0_ref, a1_ref, a2_ref, h_ref, wihT_ref, whhT_ref, bih_ref,
              bhh_ref, wnext_ref, h1_ref, m0_ref, m1_ref, m2_ref):
    hn = _gru_block(a0_ref[...], a1_ref[...], a2_ref[...], h_ref[...],
                    wihT_ref[...], whhT_ref[...], bih_ref[...], bhh_ref[...])
    h1_ref[...] = hn
    m = jnp.dot(hn, wnext_ref[...], preferred_element_type=jnp.float32)
    m0_ref[...] = m[:, 0:32]
    m1_ref[...] = m[:, 32:64]
    m2_ref[...] = m[:, 64:96]


def _tcb(a0, a1, a2, h, w_ihT, w_hhT, b_ih, b_hh, w_next):
    part_spec = pl.BlockSpec((2, BLK, 32), lambda j: (0, j, 0))
    return pl.pallas_call(
        _tcb_body,
        grid=(NBLK,),
        in_specs=[
            part_spec, part_spec, part_spec,
            pl.BlockSpec((BLK, H), lambda j: (j, 0)),
            pl.BlockSpec((H, 3 * H), lambda j: (0, 0)),
            pl.BlockSpec((H, 3 * H), lambda j: (0, 0)),
            pl.BlockSpec((1, 3 * H), lambda j: (0, 0)),
            pl.BlockSpec((1, 3 * H), lambda j: (0, 0)),
            pl.BlockSpec((H, H), lambda j: (0, 0)),
        ],
        out_specs=[
            pl.BlockSpec((BLK, H), lambda j: (j, 0)),
            pl.BlockSpec((BLK, 32), lambda j: (j, 0)),
            pl.BlockSpec((BLK, 32), lambda j: (j, 0)),
            pl.BlockSpec((BLK, 32), lambda j: (j, 0)),
        ],
        out_shape=[
            jax.ShapeDtypeStruct((N_PAD, H), jnp.float32),
            jax.ShapeDtypeStruct((N_PAD, 32), jnp.float32),
            jax.ShapeDtypeStruct((N_PAD, 32), jnp.float32),
            jax.ShapeDtypeStruct((N_PAD, 32), jnp.float32),
        ],
        interpret=_INTERPRET,
    )(a0, a1, a2, h, w_ihT, w_hhT, b_ih, b_hh, w_next)


def _tcf_body(a0_ref, a1_ref, a2_ref, h_ref, wihT_ref, whhT_ref, bih_ref,
              bhh_ref, batch_ref, bn_s_ref, bn_t_ref, fc1T_ref, fc1b_ref,
              fc2T_ref, fc2b_ref, fc3T_ref, fc3b_ref, y_ref, acc_ref):
    j = pl.program_id(0)
    hn = _gru_block(a0_ref[...], a1_ref[...], a2_ref[...], h_ref[...],
                    wihT_ref[...], whhT_ref[...], bih_ref[...], bhh_ref[...])
    hr = jax.nn.relu(hn)

    @pl.when(j == 0)
    def _():
        acc_ref[...] = jnp.zeros_like(acc_ref)

    b = batch_ref[0, 0, :]
    gid = lax.broadcasted_iota(jnp.int32, (BLKF, G), 1)
    onehot = (b[:, None] == gid).astype(jnp.float32)
    hcat = jnp.concatenate([hr, jnp.ones((BLKF, 1), jnp.float32)], axis=1)
    acc_ref[...] += lax.dot_general(
        onehot, hcat, (((0,), (0,)), ((), ())),
        preferred_element_type=jnp.float32)

    @pl.when(j == NBLKF - 1)
    def _():
        acc = acc_ref[...]
        cnt = jnp.maximum(acc[:, H:H + 1], 1.0)
        pooled = acc[:, 0:H] / cnt
        yb = pooled * bn_s_ref[...] + bn_t_ref[...]
        y1 = jax.nn.relu(
            jnp.dot(yb, fc1T_ref[...], preferred_element_type=jnp.float32)
            + fc1b_ref[...])
        y2 = jax.nn.relu(
            jnp.dot(y1, fc2T_ref[...], preferred_element_type=jnp.float32)
            + fc2b_ref[...])
        y_ref[...] = (jnp.dot(y2, fc3T_ref[...],
                              preferred_element_type=jnp.float32)
                      + fc3b_ref[...])


def _tcf(a0, a1, a2, h, w_ihT, w_hhT, b_ih, b_hh, batch3d, bn_s, bn_t,
         fc1T, fc1b, fc2T, fc2b, fc3T, fc3b):
    part_spec = pl.BlockSpec((2, BLKF, 32), lambda j: (0, j, 0))
    full = lambda shape: pl.BlockSpec(shape, lambda j: tuple(0 for _ in shape))
    return pl.pallas_call(
        _tcf_body,
        grid=(NBLKF,),
        in_specs=[
            part_spec, part_spec, part_spec,
            pl.BlockSpec((BLKF, H), lambda j: (j, 0)),
            full((H, 3 * H)), full((H, 3 * H)), full((1, 3 * H)),
            full((1, 3 * H)),
            pl.BlockSpec((1, 1, BLKF), lambda j: (j, 0, 0)),
            full((1, H)), full((1, H)),
            full((H, 4 * H)), full((1, 4 * H)),
            full((4 * H, H)), full((1, H)),
            full((H, 1)), full((1, 1)),
        ],
        out_specs=pl.BlockSpec((G, 1), lambda j: (0, 0)),
        out_shape=jax.ShapeDtypeStruct((G, 1), jnp.float32),
        scratch_shapes=[pltpu.VMEM((G, H + 1), jnp.float32)],
        interpret=_INTERPRET,
    )(a0, a1, a2, h, w_ihT, w_hhT, b_ih, b_hh, batch3d, bn_s, bn_t,
      fc1T, fc1b, fc2T, fc2b, fc3T, fc3b)


NW = 32
CLEN = 128
NSG = 50
CHUNKS = NSG * 4
EPW = CHUNKS * CLEN
E_PAD = NW * EPW
WB = 3200
N_OUT = 16 * WB
R_ACC = N_OUT + 1024
ZROWS = R_ACC // 16


def _sc_agg_kernel(m0_hbm, m1_hbm, m2_hbm, src_hbm, dst_hbm,
                   a0_hbm, a1_hbm, a2_hbm,
                   src_ib, dst_ib, rows, zbuf, acc, isem, gsem, ssem):
    c = lax.axis_index("c")
    s = lax.axis_index("s")
    wid = c * 16 + s

    zero16 = jnp.zeros((16,), jnp.float32)

    @pl.loop(0, 64)
    def _(r):
        zbuf[r, pl.ds(0, 16)] = zero16
        zbuf[r, pl.ds(16, 16)] = zero16

    def start_idx(sg, b):
        pltpu.async_copy(src_hbm.at[wid, pl.ds(sg * 4, 4)], src_ib.at[b],
                         isem.at[b])
        pltpu.async_copy(dst_hbm.at[wid, pl.ds(sg * 4, 4)], dst_ib.at[b],
                         isem.at[b])

    def wait_idx(b):
        pltpu.make_async_copy(src_hbm.at[0, pl.ds(0, 4)], src_ib.at[b],
                              isem.at[b]).wait()
        pltpu.make_async_copy(src_hbm.at[0, pl.ds(0, 4)], dst_ib.at[b],
                              isem.at[b]).wait()

    def drain_rows_sem(sem, b):
        pltpu.make_async_copy(m0_hbm.at[pl.ds(0, CLEN)], rows.at[b],
                              sem.at[b]).wait()

    for p, (mg, ag) in enumerate(((m0_hbm, a0_hbm), (m1_hbm, a1_hbm),
                                  (m2_hbm, a2_hbm))):
        @pl.loop(0, ZROWS // 64)
        def _(t):
            pltpu.sync_copy(zbuf, acc.at[pl.ds(s * ZROWS + t * 64, 64)])
        plsc.subcore_barrier()

        def process_sg(sg, bI, first):
            start_idx(sg + 1, bI ^ 1)
            wait_idx(bI)
            for q in range(4):
                if not first:
                    drain_rows_sem(ssem, q)
                pltpu.async_copy(mg.at[src_ib.at[bI, q]],
                                 rows.at[q], gsem.at[q])
            for q in range(4):
                drain_rows_sem(gsem, q)
                pltpu.async_copy(rows.at[q],
                                 acc.at[dst_ib.at[bI, q]],
                                 ssem.at[q], add=True)

        start_idx(0, 0)
        process_sg(0, 0, True)

        @pl.loop(0, (NSG - 2) // 2)
        def _(g):
            process_sg(1 + 2 * g, 1, False)
            process_sg(2 + 2 * g, 0, False)

        process_sg(NSG - 1, 1, False)

        for q in range(4):
            drain_rows_sem(ssem, q)
        wait_idx(0)
        plsc.subcore_barrier()

        pltpu.sync_copy(acc.at[pl.ds(s * WB, WB)],
                        ag.at[c, pl.ds(s * WB, WB)])
        plsc.subcore_barrier()


def _sc_agg(m0, m1, m2, srcp, dstp):
    mesh = plsc.VectorSubcoreMesh(core_axis_name="c", subcore_axis_name="s")
    f = pl.kernel(
        _sc_agg_kernel,
        out_type=[jax.ShapeDtypeStruct((2, N_OUT, 32), jnp.float32)] * 3,
        mesh=mesh,
        scratch_types=[
            pltpu.VMEM((2, 4, CLEN), jnp.int32),
            pltpu.VMEM((2, 4, CLEN), jnp.int32),
            pltpu.VMEM((4, CLEN, 32), jnp.float32),
            pltpu.VMEM((64, 32), jnp.float32),
            pltpu.VMEM_SHARED((R_ACC, 32), jnp.float32),
            pltpu.SemaphoreType.DMA((2,)),
            pltpu.SemaphoreType.DMA((4,)),
            pltpu.SemaphoreType.DMA((4,)),
        ],
        compiler_params=pltpu.CompilerParams(use_tc_tiling_on_sc=False),
    )
    return f(m0, m1, m2, srcp, dstp)


def _edge_planes(edge_index):
    src = edge_index[0]
    dst = edge_index[1]
    pad = E_PAD - E
    fill_src = (jnp.arange(pad, dtype=jnp.int32) * 997) % N
    fill_dst = N_OUT + (jnp.arange(pad, dtype=jnp.int32) % CLEN)
    junk = jnp.zeros((NW, 8, CLEN), jnp.int32)
    srcp = jnp.concatenate([src, fill_src]).reshape(NW, CHUNKS, CLEN)
    dstp = jnp.concatenate([dst, fill_dst]).reshape(NW, CHUNKS, CLEN)
    srcp = jnp.concatenate([srcp, junk], axis=1)
    dstp = jnp.concatenate([dstp, junk], axis=1)
    return srcp, dstp


def kernel(x, edge_index, batch, ggc_weight, w_ih, w_hh, b_ih, b_hh,
           bn_gamma, bn_beta, bn_mean, bn_var,
           fc1_w, fc1_b, fc2_w, fc2_b, fc3_w, fc3_b):
    w_ihT = w_ih.T
    w_hhT = w_hh.T
    b_ih2 = b_ih[None, :]
    b_hh2 = b_hh[None, :]
    inv = bn_gamma / jnp.sqrt(bn_var + 1e-5)
    bn_s = inv[None, :]
    bn_t = (bn_beta - bn_mean * inv)[None, :]
    fc1T = fc1_w.T
    fc2T = fc2_w.T
    fc3T = fc3_w.T
    fc1b = fc1_b[None, :]
    fc2b = fc2_b[None, :]
    fc3b = fc3_b[None, :]
    x_p = jnp.pad(x, ((0, N_PAD - N), (0, 0)))
    batch_p = jnp.pad(batch, (0, N_PAD - N), constant_values=G)
    batch3d = batch_p.reshape(NBLKF, 1, BLKF)

    srcp, dstp = _edge_planes(edge_index)

    m0, m1, m2, h = _tc0(x_p, ggc_weight[0][:F_IN, :])
    for i in range(L):
        a0, a1, a2 = _sc_agg(m0, m1, m2, srcp, dstp)
        if i < L - 1:
            h, m0, m1, m2 = _tcb(a0, a1, a2, h, w_ihT, w_hhT, b_ih2, b_hh2,
                                 ggc_weight[i + 1])
        else:
            y = _tcf(a0, a1, a2, h, w_ihT, w_hhT, b_ih2, b_hh2, batch3d,
                     bn_s, bn_t, fc1T, fc1b, fc2T, fc2b, fc3T, fc3b)
    return y
